# Initial kernel scaffold; baseline (speedup 1.0000x reference)
import functools

import jax
import jax.numpy as jnp
from jax import lax
from jax.experimental import pallas as pl
from jax.experimental.pallas import tpu as pltpu
from jax.experimental.pallas import tpu_sc as plsc

N_NODES = 10000
IN_CH = 128
OUT_CH = 128
N_EDGES = 320000

NC = 2
NS = 16
NW = NC * NS

CHUNK = 80
N_CHUNKS = N_EDGES // CHUNK
CHUNKS_PER_TILE = N_CHUNKS // NW
ROW_CHUNKS = N_NODES // CHUNK


def _sc_body(x_hbm, src_hbm, dst_hbm, agg_out, deg_out,
             src_v, dst_v, rows_v, ones_v, zrows_v, zvec_v,
             agg_sh, deg_sh, sem):
    c = lax.axis_index("c")
    s = lax.axis_index("s")
    wid = s * NC + c

    z16 = jnp.zeros((16,), jnp.float32)
    o16 = jnp.ones((16,), jnp.float32)
    for j in range(CHUNK // 16):
        ones_v[pl.ds(j * 16, 16)] = o16
        zvec_v[pl.ds(j * 16, 16)] = z16

    def zrow_body(i, carry):
        for j in range(IN_CH // 16):
            zrows_v[i, pl.ds(j * 16, 16)] = z16
        return carry
    lax.fori_loop(0, CHUNK, zrow_body, 0)

    for i in range(-(-ROW_CHUNKS // NW)):
        k = wid + i * NW

        @pl.when(k < ROW_CHUNKS)
        def _():
            pltpu.sync_copy(zrows_v, agg_sh.at[pl.ds(k * CHUNK, CHUNK)])
            pltpu.sync_copy(zvec_v, deg_sh.at[pl.ds(k * CHUNK, CHUNK)])

    plsc.subcore_barrier()

    def edge_body(i, carry):
        off = (wid + i * NW) * CHUNK
        pltpu.sync_copy(src_hbm.at[pl.ds(off, CHUNK)], src_v)
        pltpu.sync_copy(dst_hbm.at[pl.ds(off, CHUNK)], dst_v)
        pltpu.async_copy(x_hbm.at[src_v], rows_v, sem).wait()
        pltpu.sync_copy(rows_v, agg_sh.at[dst_v], add=True)
        pltpu.sync_copy(ones_v, deg_sh.at[dst_v], add=True)
        return carry
    lax.fori_loop(0, CHUNKS_PER_TILE, edge_body, 0)

    plsc.subcore_barrier()

    for i in range(-(-ROW_CHUNKS // NS)):
        k = s + i * NS

        @pl.when(k < ROW_CHUNKS)
        def _():
            pltpu.sync_copy(agg_sh.at[pl.ds(k * CHUNK, CHUNK)],
                            agg_out.at[c, pl.ds(k * CHUNK, CHUNK)])
            pltpu.sync_copy(deg_sh.at[pl.ds(k * CHUNK, CHUNK)],
                            deg_out.at[c, pl.ds(k * CHUNK, CHUNK)])


_sc_aggregate = functools.partial(
    pl.kernel,
    out_type=(jax.ShapeDtypeStruct((NC, N_NODES, IN_CH), jnp.float32),
              jax.ShapeDtypeStruct((NC, N_NODES), jnp.float32)),
    mesh=plsc.VectorSubcoreMesh(core_axis_name="c", subcore_axis_name="s"),
    scratch_types=[
        pltpu.VMEM((CHUNK,), jnp.int32),
        pltpu.VMEM((CHUNK,), jnp.int32),
        pltpu.VMEM((CHUNK, IN_CH), jnp.float32),
        pltpu.VMEM((CHUNK,), jnp.float32),
        pltpu.VMEM((CHUNK, IN_CH), jnp.float32),
        pltpu.VMEM((CHUNK,), jnp.float32),
        pltpu.VMEM_SHARED((N_NODES, IN_CH), jnp.float32),
        pltpu.VMEM_SHARED((N_NODES,), jnp.float32),
        pltpu.SemaphoreType.DMA,
    ],
)(_sc_body)


ROW_BLK = 1000


def _tc_body(agg_ref, deg_ref, wmsgT_ref, bmsg_ref, wupdT_ref, bupd_ref,
             out_ref):
    a = agg_ref[0] + agg_ref[1]
    d = deg_ref[0] + deg_ref[1]
    m = jnp.dot(a, wmsgT_ref[...], preferred_element_type=jnp.float32)
    m = m + d * bmsg_ref[...]
    u = jnp.dot(m, wupdT_ref[...], preferred_element_type=jnp.float32)
    out_ref[...] = u + bupd_ref[...]


_tc_update = pl.pallas_call(
    _tc_body,
    grid=(N_NODES // ROW_BLK,),
    in_specs=[
        pl.BlockSpec((NC, ROW_BLK, IN_CH), lambda i: (0, i, 0)),
        pl.BlockSpec((NC, ROW_BLK, 1), lambda i: (0, i, 0)),
        pl.BlockSpec((IN_CH, OUT_CH), lambda i: (0, 0)),
        pl.BlockSpec((1, OUT_CH), lambda i: (0, 0)),
        pl.BlockSpec((OUT_CH, OUT_CH), lambda i: (0, 0)),
        pl.BlockSpec((1, OUT_CH), lambda i: (0, 0)),
    ],
    out_specs=pl.BlockSpec((ROW_BLK, OUT_CH), lambda i: (i, 0)),
    out_shape=jax.ShapeDtypeStruct((N_NODES, OUT_CH), jnp.float32),
)


@jax.jit
def kernel(x, edge_index, W_msg, b_msg, W_upd, b_upd):
    src = edge_index[0].astype(jnp.int32)
    dst = edge_index[1].astype(jnp.int32)
    agg, deg = _sc_aggregate(x, src, dst)
    return _tc_update(agg, deg.reshape(NC, N_NODES, 1),
                      W_msg.T, b_msg.reshape(1, OUT_CH),
                      W_upd.T, b_upd.reshape(1, OUT_CH))


# speedup vs baseline: 5.2043x; 5.2043x over previous
import functools

import jax
import jax.numpy as jnp
from jax import lax
from jax.experimental import pallas as pl
from jax.experimental.pallas import tpu as pltpu
from jax.experimental.pallas import tpu_sc as plsc

N_NODES = 10000
IN_CH = 128
OUT_CH = 128
N_EDGES = 320000

NC = 2
NS = 16
NW = NC * NS

CHUNK = 80
N_CHUNKS = N_EDGES // CHUNK
CHUNKS_PER_TILE = N_CHUNKS // NW
ROW_CHUNKS = N_NODES // CHUNK


def _sc_body(x_hbm, src_hbm, dst_hbm, agg_out, deg_out,
             src_v, dst_v, rows_v, ones_v, zrows_v, zvec_v,
             agg_sh, deg_sh, sem):
    c = lax.axis_index("c")
    s = lax.axis_index("s")
    wid = s * NC + c

    z16 = jnp.zeros((16,), jnp.float32)
    o16 = jnp.ones((16,), jnp.float32)
    for j in range(CHUNK // 16):
        ones_v[pl.ds(j * 16, 16)] = o16
        zvec_v[pl.ds(j * 16, 16)] = z16

    def zrow_body(i, carry):
        for j in range(IN_CH // 16):
            zrows_v[i, pl.ds(j * 16, 16)] = z16
        return carry
    lax.fori_loop(0, CHUNK, zrow_body, 0)

    for i in range(-(-ROW_CHUNKS // NS)):
        k = s + i * NS

        @pl.when(k < ROW_CHUNKS)
        def _():
            pltpu.sync_copy(zrows_v, agg_sh.at[pl.ds(k * CHUNK, CHUNK)])
            pltpu.sync_copy(zvec_v, deg_sh.at[pl.ds(k * CHUNK, CHUNK)])

    plsc.subcore_barrier()

    def edge_body(i, carry):
        off = (wid + i * NW) * CHUNK
        pltpu.sync_copy(src_hbm.at[pl.ds(off, CHUNK)], src_v)
        pltpu.sync_copy(dst_hbm.at[pl.ds(off, CHUNK)], dst_v)
        pltpu.async_copy(x_hbm.at[src_v], rows_v, sem).wait()
        pltpu.sync_copy(rows_v, agg_sh.at[dst_v], add=True)
        pltpu.sync_copy(ones_v, deg_sh.at[dst_v], add=True)
        return carry
    lax.fori_loop(0, CHUNKS_PER_TILE, edge_body, 0)

    plsc.subcore_barrier()

    for i in range(-(-ROW_CHUNKS // NS)):
        k = s + i * NS

        @pl.when(k < ROW_CHUNKS)
        def _():
            pltpu.sync_copy(agg_sh.at[pl.ds(k * CHUNK, CHUNK)], rows_v)
            pltpu.sync_copy(rows_v, agg_out.at[c, pl.ds(k * CHUNK, CHUNK)])
            pltpu.sync_copy(deg_sh.at[pl.ds(k * CHUNK, CHUNK)], zvec_v)
            pltpu.sync_copy(zvec_v,
                            deg_out.at[pl.ds(c * N_NODES + k * CHUNK, CHUNK)])


_sc_aggregate = functools.partial(
    pl.kernel,
    out_type=(jax.ShapeDtypeStruct((NC, N_NODES, IN_CH), jnp.float32),
              jax.ShapeDtypeStruct((NC * N_NODES,), jnp.float32)),
    mesh=plsc.VectorSubcoreMesh(core_axis_name="c", subcore_axis_name="s"),
    scratch_types=[
        pltpu.VMEM((CHUNK,), jnp.int32),
        pltpu.VMEM((CHUNK,), jnp.int32),
        pltpu.VMEM((CHUNK, IN_CH), jnp.float32),
        pltpu.VMEM((CHUNK,), jnp.float32),
        pltpu.VMEM((CHUNK, IN_CH), jnp.float32),
        pltpu.VMEM((CHUNK,), jnp.float32),
        pltpu.VMEM_SHARED((N_NODES, IN_CH), jnp.float32),
        pltpu.VMEM_SHARED((N_NODES,), jnp.float32),
        pltpu.SemaphoreType.DMA,
    ],
)(_sc_body)


ROW_BLK = 1000


def _tc_body(agg_ref, deg_ref, wmsgT_ref, bmsg_ref, wupdT_ref, bupd_ref,
             out_ref):
    a = agg_ref[0] + agg_ref[1]
    d = deg_ref[0] + deg_ref[1]
    m = jnp.dot(a, wmsgT_ref[...], preferred_element_type=jnp.float32)
    m = m + d * bmsg_ref[...]
    u = jnp.dot(m, wupdT_ref[...], preferred_element_type=jnp.float32)
    out_ref[...] = u + bupd_ref[...]


_tc_update = pl.pallas_call(
    _tc_body,
    grid=(N_NODES // ROW_BLK,),
    in_specs=[
        pl.BlockSpec((NC, ROW_BLK, IN_CH), lambda i: (0, i, 0)),
        pl.BlockSpec((NC, ROW_BLK, 1), lambda i: (0, i, 0)),
        pl.BlockSpec((IN_CH, OUT_CH), lambda i: (0, 0)),
        pl.BlockSpec((1, OUT_CH), lambda i: (0, 0)),
        pl.BlockSpec((OUT_CH, OUT_CH), lambda i: (0, 0)),
        pl.BlockSpec((1, OUT_CH), lambda i: (0, 0)),
    ],
    out_specs=pl.BlockSpec((ROW_BLK, OUT_CH), lambda i: (i, 0)),
    out_shape=jax.ShapeDtypeStruct((N_NODES, OUT_CH), jnp.float32),
)


@jax.jit
def kernel(x, edge_index, W_msg, b_msg, W_upd, b_upd):
    src = edge_index[0].astype(jnp.int32)
    dst = edge_index[1].astype(jnp.int32)
    agg, deg = _sc_aggregate(x, src, dst)
    return _tc_update(agg, deg.reshape(NC, N_NODES, 1),
                      W_msg.T, b_msg.reshape(1, OUT_CH),
                      W_upd.T, b_upd.reshape(1, OUT_CH))


# speedup vs baseline: 10.6632x; 2.0489x over previous
import functools

import jax
import jax.numpy as jnp
from jax import lax
from jax.experimental import pallas as pl
from jax.experimental.pallas import tpu as pltpu
from jax.experimental.pallas import tpu_sc as plsc

N_NODES = 10000
IN_CH = 128
OUT_CH = 128
N_EDGES = 320000

NC = 2
NS = 16
NW = NC * NS

CHUNK = 40
N_CHUNKS = N_EDGES // CHUNK
CPT = N_CHUNKS // NW
GRP = 5
N_GRP = CPT // GRP
NIDX = 8
ROW_CHUNKS = N_NODES // CHUNK
WB_STEPS = -(-ROW_CHUNKS // NS)


def _sc_body(x_hbm, src_hbm, dst_hbm, agg_out, deg_out,
             src_i, dst_i, rows_v, ones_v, zvec_v, degb_v,
             agg_sh, deg_sh, sg, ss, sd, si):
    c = lax.axis_index("c")
    s = lax.axis_index("s")
    wid = s * NC + c
    base = wid * (CPT * CHUNK)

    def idx_load(cidx, slot):
        pltpu.async_copy(src_hbm.at[pl.ds(base + cidx * CHUNK, CHUNK)],
                         src_i.at[slot], si)
        pltpu.async_copy(dst_hbm.at[pl.ds(base + cidx * CHUNK, CHUNK)],
                         dst_i.at[slot], si)

    def idx_wait():
        for _ in range(2):
            pltpu.make_async_copy(src_hbm.at[pl.ds(0, CHUNK)],
                                  src_i.at[0], si).wait()

    def gather(cidx, islot, rslot):
        pltpu.async_copy(x_hbm.at[src_i.at[islot]], rows_v.at[rslot], sg)

    def gather_wait():
        pltpu.make_async_copy(x_hbm.at[src_i.at[0]], rows_v.at[0], sg).wait()

    def scatter(islot, rslot):
        pltpu.async_copy(rows_v.at[rslot], agg_sh.at[dst_i.at[islot]],
                         ss, add=True)
        pltpu.async_copy(ones_v.at[pl.ds(0, CHUNK)],
                         deg_sh.at[dst_i.at[islot]], sd, add=True)

    def scatter_wait():
        pltpu.make_async_copy(rows_v.at[0], agg_sh.at[dst_i.at[0]],
                              ss).wait()
        pltpu.make_async_copy(ones_v.at[pl.ds(0, CHUNK)],
                              deg_sh.at[dst_i.at[0]], sd).wait()

    z16 = jnp.zeros((16,), jnp.float32)
    o16 = jnp.ones((16,), jnp.float32)
    for j in range(3):
        ones_v[pl.ds(j * 16, 16)] = o16
        zvec_v[pl.ds(j * 16, 16)] = z16

    def zrow_body(i, carry):
        for j in range(IN_CH // 16):
            rows_v[0, i, pl.ds(j * 16, 16)] = z16
        return carry
    lax.fori_loop(0, CHUNK, zrow_body, 0)

    for j in range(4):
        idx_load(j, j)

    for i in range(WB_STEPS):
        k = s + i * NS

        @pl.when(k < ROW_CHUNKS)
        def _():
            pltpu.async_copy(rows_v.at[0], agg_sh.at[pl.ds(k * CHUNK, CHUNK)],
                             ss)
            pltpu.async_copy(zvec_v.at[pl.ds(0, CHUNK)],
                             deg_sh.at[pl.ds(k * CHUNK, CHUNK)], sd)
    for i in range(WB_STEPS):
        k = s + i * NS

        @pl.when(k < ROW_CHUNKS)
        def _():
            pltpu.make_async_copy(rows_v.at[0],
                                  agg_sh.at[pl.ds(k * CHUNK, CHUNK)],
                                  ss).wait()
            pltpu.make_async_copy(zvec_v.at[pl.ds(0, CHUNK)],
                                  deg_sh.at[pl.ds(k * CHUNK, CHUNK)],
                                  sd).wait()

    for j in range(2):
        idx_wait()
        gather(j, j, j)

    plsc.subcore_barrier()

    def main_body(t, carry):
        for b in range(GRP):
            i = t * GRP + b

            @pl.when(i + 4 < CPT)
            def _():
                idx_load(i + 4, lax.rem(i + 4, NIDX))

            gather_wait()
            scatter(lax.rem(i, NIDX), b)

            @pl.when(i >= 3)
            def _():
                scatter_wait()

            @pl.when(i + 2 < CPT)
            def _():
                idx_wait()
                gather(i + 2, lax.rem(i + 2, NIDX), (b + 2) % GRP)
        return carry
    lax.fori_loop(0, N_GRP, main_body, 0)

    for _ in range(3):
        scatter_wait()

    plsc.subcore_barrier()

    def wb_in(i):
        k = s + i * NS
        pltpu.async_copy(agg_sh.at[pl.ds(k * CHUNK, CHUNK)],
                         rows_v.at[i % GRP], sg)
        pltpu.async_copy(deg_sh.at[pl.ds(k * CHUNK, CHUNK)],
                         degb_v.at[i, pl.ds(0, CHUNK)], si)

    def wb_in_wait(i):
        k = s + i * NS
        pltpu.make_async_copy(agg_sh.at[pl.ds(k * CHUNK, CHUNK)],
                              rows_v.at[i % GRP], sg).wait()
        pltpu.make_async_copy(deg_sh.at[pl.ds(k * CHUNK, CHUNK)],
                              degb_v.at[i, pl.ds(0, CHUNK)], si).wait()

    def wb_out(i):
        k = s + i * NS
        pltpu.async_copy(rows_v.at[i % GRP],
                         agg_out.at[c, pl.ds(k * CHUNK, CHUNK)], ss)
        pltpu.async_copy(degb_v.at[i, pl.ds(0, CHUNK)],
                         deg_out.at[pl.ds(c * N_NODES + k * CHUNK, CHUNK)],
                         sd)

    def wb_out_wait(i):
        k = s + i * NS
        pltpu.make_async_copy(rows_v.at[i % GRP],
                              agg_out.at[c, pl.ds(k * CHUNK, CHUNK)],
                              ss).wait()
        pltpu.make_async_copy(
            degb_v.at[i, pl.ds(0, CHUNK)],
            deg_out.at[pl.ds(c * N_NODES + k * CHUNK, CHUNK)], sd).wait()

    last_ok = s + (WB_STEPS - 1) * NS < ROW_CHUNKS
    for i in range(WB_STEPS):
        if i >= GRP:
            wb_out_wait(i - GRP)
        if i < WB_STEPS - 1:
            wb_in(i)
        else:
            @pl.when(last_ok)
            def _():
                wb_in(i)
        if i >= 1:
            wb_in_wait(i - 1)
            wb_out(i - 1)

    @pl.when(last_ok)
    def _():
        wb_in_wait(WB_STEPS - 1)
        wb_out(WB_STEPS - 1)
    for i in range(WB_STEPS - GRP, WB_STEPS - 1):
        wb_out_wait(i)

    @pl.when(last_ok)
    def _():
        wb_out_wait(WB_STEPS - 1)


_sc_aggregate = functools.partial(
    pl.kernel,
    out_type=(jax.ShapeDtypeStruct((NC, N_NODES, IN_CH), jnp.float32),
              jax.ShapeDtypeStruct((NC * N_NODES,), jnp.float32)),
    mesh=plsc.VectorSubcoreMesh(core_axis_name="c", subcore_axis_name="s"),
    scratch_types=[
        pltpu.VMEM((NIDX, CHUNK), jnp.int32),
        pltpu.VMEM((NIDX, CHUNK), jnp.int32),
        pltpu.VMEM((GRP, CHUNK, IN_CH), jnp.float32),
        pltpu.VMEM((48,), jnp.float32),
        pltpu.VMEM((48,), jnp.float32),
        pltpu.VMEM((WB_STEPS, 48), jnp.float32),
        pltpu.VMEM_SHARED((N_NODES, IN_CH), jnp.float32),
        pltpu.VMEM_SHARED((N_NODES,), jnp.float32),
        pltpu.SemaphoreType.DMA,
        pltpu.SemaphoreType.DMA,
        pltpu.SemaphoreType.DMA,
        pltpu.SemaphoreType.DMA,
    ],
)(_sc_body)


ROW_BLK = 1000


def _tc_body(agg_ref, deg_ref, wmsgT_ref, bmsg_ref, wupdT_ref, bupd_ref,
             out_ref):
    a = agg_ref[0] + agg_ref[1]
    d = deg_ref[0] + deg_ref[1]
    m = jnp.dot(a, wmsgT_ref[...], preferred_element_type=jnp.float32)
    m = m + d * bmsg_ref[...]
    u = jnp.dot(m, wupdT_ref[...], preferred_element_type=jnp.float32)
    out_ref[...] = u + bupd_ref[...]


_tc_update = pl.pallas_call(
    _tc_body,
    grid=(N_NODES // ROW_BLK,),
    in_specs=[
        pl.BlockSpec((NC, ROW_BLK, IN_CH), lambda i: (0, i, 0)),
        pl.BlockSpec((NC, ROW_BLK, 1), lambda i: (0, i, 0)),
        pl.BlockSpec((IN_CH, OUT_CH), lambda i: (0, 0)),
        pl.BlockSpec((1, OUT_CH), lambda i: (0, 0)),
        pl.BlockSpec((OUT_CH, OUT_CH), lambda i: (0, 0)),
        pl.BlockSpec((1, OUT_CH), lambda i: (0, 0)),
    ],
    out_specs=pl.BlockSpec((ROW_BLK, OUT_CH), lambda i: (i, 0)),
    out_shape=jax.ShapeDtypeStruct((N_NODES, OUT_CH), jnp.float32),
)


@jax.jit
def kernel(x, edge_index, W_msg, b_msg, W_upd, b_upd):
    src = edge_index[0].astype(jnp.int32)
    dst = edge_index[1].astype(jnp.int32)
    agg, deg = _sc_aggregate(x, src, dst)
    return _tc_update(agg, deg.reshape(NC, N_NODES, 1),
                      W_msg.T, b_msg.reshape(1, OUT_CH),
                      W_upd.T, b_upd.reshape(1, OUT_CH))


# speedup vs baseline: 13.3861x; 1.2554x over previous
import functools

import jax
import jax.numpy as jnp
from jax import lax
from jax.experimental import pallas as pl
from jax.experimental.pallas import tpu as pltpu
from jax.experimental.pallas import tpu_sc as plsc

N_NODES = 10000
IN_CH = 128
OUT_CH = 128
N_EDGES = 320000

NC = 2
NS = 16
NW = NC * NS

CHUNK = 40
N_CHUNKS = N_EDGES // CHUNK
CPT = N_CHUNKS // NW
GRP = 5
RING = 7
GDEPTH = 4
N_GRP = CPT // GRP
NIDX = 16
ROW_CHUNKS = N_NODES // CHUNK
WB_STEPS = -(-ROW_CHUNKS // NS)


def _sc_body(x_hbm, src_hbm, dst_hbm, agg_out, deg_out,
             src_i, dst_i, rows_v, ones_v, zvec_v, degb_v,
             agg_sh, deg_sh, sg, ss, sd, si):
    c = lax.axis_index("c")
    s = lax.axis_index("s")
    wid = s * NC + c
    base = wid * (CPT * CHUNK)

    def idx_load(cidx, slot):
        pltpu.async_copy(src_hbm.at[pl.ds(base + cidx * CHUNK, CHUNK)],
                         src_i.at[slot], si)
        pltpu.async_copy(dst_hbm.at[pl.ds(base + cidx * CHUNK, CHUNK)],
                         dst_i.at[slot], si)

    def idx_wait():
        for _ in range(2):
            pltpu.make_async_copy(src_hbm.at[pl.ds(0, CHUNK)],
                                  src_i.at[0], si).wait()

    def gather(cidx, islot, rslot):
        pltpu.async_copy(x_hbm.at[src_i.at[islot]], rows_v.at[rslot], sg)

    def gather_wait():
        pltpu.make_async_copy(x_hbm.at[src_i.at[0]], rows_v.at[0], sg).wait()

    def scatter(islot, rslot):
        pltpu.async_copy(rows_v.at[rslot], agg_sh.at[dst_i.at[islot]],
                         ss, add=True)
        pltpu.async_copy(ones_v.at[pl.ds(0, CHUNK)],
                         deg_sh.at[dst_i.at[islot]], sd, add=True)

    def scatter_wait():
        pltpu.make_async_copy(rows_v.at[0], agg_sh.at[dst_i.at[0]],
                              ss).wait()
        pltpu.make_async_copy(ones_v.at[pl.ds(0, CHUNK)],
                              deg_sh.at[dst_i.at[0]], sd).wait()

    z16 = jnp.zeros((16,), jnp.float32)
    o16 = jnp.ones((16,), jnp.float32)
    for j in range(3):
        ones_v[pl.ds(j * 16, 16)] = o16
        zvec_v[pl.ds(j * 16, 16)] = z16

    def zrow_body(i, carry):
        for j in range(IN_CH // 16):
            rows_v[0, i, pl.ds(j * 16, 16)] = z16
        return carry
    lax.fori_loop(0, CHUNK, zrow_body, 0)

    for j in range(GDEPTH + 2):
        idx_load(j, j)

    for i in range(WB_STEPS):
        k = s + i * NS

        @pl.when(k < ROW_CHUNKS)
        def _():
            pltpu.async_copy(rows_v.at[0], agg_sh.at[pl.ds(k * CHUNK, CHUNK)],
                             ss)
            pltpu.async_copy(zvec_v.at[pl.ds(0, CHUNK)],
                             deg_sh.at[pl.ds(k * CHUNK, CHUNK)], sd)
    for i in range(WB_STEPS):
        k = s + i * NS

        @pl.when(k < ROW_CHUNKS)
        def _():
            pltpu.make_async_copy(rows_v.at[0],
                                  agg_sh.at[pl.ds(k * CHUNK, CHUNK)],
                                  ss).wait()
            pltpu.make_async_copy(zvec_v.at[pl.ds(0, CHUNK)],
                                  deg_sh.at[pl.ds(k * CHUNK, CHUNK)],
                                  sd).wait()

    for j in range(GDEPTH):
        idx_wait()
        gather(j, j, j)

    plsc.subcore_barrier()

    def main_body(t, carry):
        for b in range(GRP):
            i = t * GRP + b

            @pl.when(i + GDEPTH + 2 < CPT)
            def _():
                idx_load(i + GDEPTH + 2, lax.rem(i + GDEPTH + 2, NIDX))

            gather_wait()
            scatter(lax.rem(i, NIDX), lax.rem(i, RING))

            @pl.when(i >= 3)
            def _():
                scatter_wait()

            @pl.when(i + GDEPTH < CPT)
            def _():
                idx_wait()
                gather(i + GDEPTH, lax.rem(i + GDEPTH, NIDX),
                       lax.rem(i + GDEPTH, RING))
        return carry
    lax.fori_loop(0, N_GRP, main_body, 0)

    for _ in range(3):
        scatter_wait()

    plsc.subcore_barrier()

    def wb_in(i):
        k = s + i * NS
        pltpu.async_copy(agg_sh.at[pl.ds(k * CHUNK, CHUNK)],
                         rows_v.at[i % RING], sg)
        pltpu.async_copy(deg_sh.at[pl.ds(k * CHUNK, CHUNK)],
                         degb_v.at[i, pl.ds(0, CHUNK)], si)

    def wb_in_wait(i):
        k = s + i * NS
        pltpu.make_async_copy(agg_sh.at[pl.ds(k * CHUNK, CHUNK)],
                              rows_v.at[i % RING], sg).wait()
        pltpu.make_async_copy(deg_sh.at[pl.ds(k * CHUNK, CHUNK)],
                              degb_v.at[i, pl.ds(0, CHUNK)], si).wait()

    def wb_out(i):
        k = s + i * NS
        pltpu.async_copy(rows_v.at[i % RING],
                         agg_out.at[c, pl.ds(k * CHUNK, CHUNK)], ss)
        pltpu.async_copy(degb_v.at[i, pl.ds(0, CHUNK)],
                         deg_out.at[pl.ds(c * N_NODES + k * CHUNK, CHUNK)],
                         sd)

    def wb_out_wait(i):
        k = s + i * NS
        pltpu.make_async_copy(rows_v.at[i % RING],
                              agg_out.at[c, pl.ds(k * CHUNK, CHUNK)],
                              ss).wait()
        pltpu.make_async_copy(
            degb_v.at[i, pl.ds(0, CHUNK)],
            deg_out.at[pl.ds(c * N_NODES + k * CHUNK, CHUNK)], sd).wait()

    last_ok = s + (WB_STEPS - 1) * NS < ROW_CHUNKS
    for i in range(WB_STEPS):
        if i >= RING:
            wb_out_wait(i - RING)
        if i < WB_STEPS - 1:
            wb_in(i)
        else:
            @pl.when(last_ok)
            def _():
                wb_in(i)
        if i >= 1:
            wb_in_wait(i - 1)
            wb_out(i - 1)

    @pl.when(last_ok)
    def _():
        wb_in_wait(WB_STEPS - 1)
        wb_out(WB_STEPS - 1)
    for i in range(WB_STEPS - RING, WB_STEPS - 1):
        wb_out_wait(i)

    @pl.when(last_ok)
    def _():
        wb_out_wait(WB_STEPS - 1)


_sc_aggregate = functools.partial(
    pl.kernel,
    out_type=(jax.ShapeDtypeStruct((NC, N_NODES, IN_CH), jnp.float32),
              jax.ShapeDtypeStruct((NC * N_NODES,), jnp.float32)),
    mesh=plsc.VectorSubcoreMesh(core_axis_name="c", subcore_axis_name="s"),
    scratch_types=[
        pltpu.VMEM((NIDX, CHUNK), jnp.int32),
        pltpu.VMEM((NIDX, CHUNK), jnp.int32),
        pltpu.VMEM((RING, CHUNK, IN_CH), jnp.float32),
        pltpu.VMEM((48,), jnp.float32),
        pltpu.VMEM((48,), jnp.float32),
        pltpu.VMEM((WB_STEPS, 48), jnp.float32),
        pltpu.VMEM_SHARED((N_NODES, IN_CH), jnp.float32),
        pltpu.VMEM_SHARED((N_NODES,), jnp.float32),
        pltpu.SemaphoreType.DMA,
        pltpu.SemaphoreType.DMA,
        pltpu.SemaphoreType.DMA,
        pltpu.SemaphoreType.DMA,
    ],
)(_sc_body)


ROW_BLK = 1000


def _tc_body(agg_ref, deg_ref, wmsgT_ref, bmsg_ref, wupdT_ref, bupd_ref,
             out_ref):
    a = agg_ref[0] + agg_ref[1]
    d = deg_ref[0] + deg_ref[1]
    m = jnp.dot(a, wmsgT_ref[...], preferred_element_type=jnp.float32)
    m = m + d * bmsg_ref[...]
    u = jnp.dot(m, wupdT_ref[...], preferred_element_type=jnp.float32)
    out_ref[...] = u + bupd_ref[...]


_tc_update = pl.pallas_call(
    _tc_body,
    grid=(N_NODES // ROW_BLK,),
    in_specs=[
        pl.BlockSpec((NC, ROW_BLK, IN_CH), lambda i: (0, i, 0)),
        pl.BlockSpec((NC, ROW_BLK, 1), lambda i: (0, i, 0)),
        pl.BlockSpec((IN_CH, OUT_CH), lambda i: (0, 0)),
        pl.BlockSpec((1, OUT_CH), lambda i: (0, 0)),
        pl.BlockSpec((OUT_CH, OUT_CH), lambda i: (0, 0)),
        pl.BlockSpec((1, OUT_CH), lambda i: (0, 0)),
    ],
    out_specs=pl.BlockSpec((ROW_BLK, OUT_CH), lambda i: (i, 0)),
    out_shape=jax.ShapeDtypeStruct((N_NODES, OUT_CH), jnp.float32),
)


@jax.jit
def kernel(x, edge_index, W_msg, b_msg, W_upd, b_upd):
    src = edge_index[0].astype(jnp.int32)
    dst = edge_index[1].astype(jnp.int32)
    agg, deg = _sc_aggregate(x, src, dst)
    return _tc_update(agg, deg.reshape(NC, N_NODES, 1),
                      W_msg.T, b_msg.reshape(1, OUT_CH),
                      W_upd.T, b_upd.reshape(1, OUT_CH))


# speedup vs baseline: 14.4714x; 1.0811x over previous
import functools

import jax
import jax.numpy as jnp
from jax import lax
from jax.experimental import pallas as pl
from jax.experimental.pallas import tpu as pltpu
from jax.experimental.pallas import tpu_sc as plsc

N_NODES = 10000
IN_CH = 128
OUT_CH = 128
N_EDGES = 320000

NC = 2
NS = 16
NW = NC * NS

CHUNK = 40
N_CHUNKS = N_EDGES // CHUNK
CPT = N_CHUNKS // NW
GRP = 5
RING = 8
GDEPTH = 5
N_GRP = CPT // GRP
NIDX = 16
ROW_CHUNKS = N_NODES // CHUNK
WB_STEPS = -(-ROW_CHUNKS // NS)


def _sc_body(x_hbm, ei_hbm, agg_out, deg_out,
             src_i, dst_i, rows_v, ones_v, zvec_v, degb_v,
             agg_sh, deg_sh, sg, ss, sd, si):
    c = lax.axis_index("c")
    s = lax.axis_index("s")
    wid = s * NC + c
    base = wid * (CPT * CHUNK)

    def idx_load(cidx, slot):
        pltpu.async_copy(ei_hbm.at[pl.ds(base + cidx * CHUNK, CHUNK)],
                         src_i.at[slot], si)
        pltpu.async_copy(
            ei_hbm.at[pl.ds(N_EDGES + base + cidx * CHUNK, CHUNK)],
            dst_i.at[slot], si)

    def idx_wait():
        for _ in range(2):
            pltpu.make_async_copy(ei_hbm.at[pl.ds(0, CHUNK)],
                                  src_i.at[0], si).wait()

    def gather(cidx, islot, rslot):
        pltpu.async_copy(x_hbm.at[src_i.at[islot]], rows_v.at[rslot], sg)

    def gather_wait():
        pltpu.make_async_copy(x_hbm.at[src_i.at[0]], rows_v.at[0], sg).wait()

    def scatter(islot, rslot):
        pltpu.async_copy(rows_v.at[rslot], agg_sh.at[dst_i.at[islot]],
                         ss, add=True)
        pltpu.async_copy(ones_v.at[pl.ds(0, CHUNK)],
                         deg_sh.at[dst_i.at[islot]], sd, add=True)

    def scatter_wait():
        pltpu.make_async_copy(rows_v.at[0], agg_sh.at[dst_i.at[0]],
                              ss).wait()
        pltpu.make_async_copy(ones_v.at[pl.ds(0, CHUNK)],
                              deg_sh.at[dst_i.at[0]], sd).wait()

    z16 = jnp.zeros((16,), jnp.float32)
    o16 = jnp.ones((16,), jnp.float32)
    for j in range(3):
        ones_v[pl.ds(j * 16, 16)] = o16
        zvec_v[pl.ds(j * 16, 16)] = z16

    def zrow_body(i, carry):
        for j in range(IN_CH // 16):
            rows_v[0, i, pl.ds(j * 16, 16)] = z16
        return carry
    lax.fori_loop(0, CHUNK, zrow_body, 0)

    for j in range(GDEPTH + 2):
        idx_load(j, j)

    for i in range(WB_STEPS):
        k = s + i * NS

        @pl.when(k < ROW_CHUNKS)
        def _():
            pltpu.async_copy(rows_v.at[0], agg_sh.at[pl.ds(k * CHUNK, CHUNK)],
                             ss)
            pltpu.async_copy(zvec_v.at[pl.ds(0, CHUNK)],
                             deg_sh.at[pl.ds(k * CHUNK, CHUNK)], sd)
    for i in range(WB_STEPS):
        k = s + i * NS

        @pl.when(k < ROW_CHUNKS)
        def _():
            pltpu.make_async_copy(rows_v.at[0],
                                  agg_sh.at[pl.ds(k * CHUNK, CHUNK)],
                                  ss).wait()
            pltpu.make_async_copy(zvec_v.at[pl.ds(0, CHUNK)],
                                  deg_sh.at[pl.ds(k * CHUNK, CHUNK)],
                                  sd).wait()

    for j in range(GDEPTH):
        idx_wait()
        gather(j, j, j)

    plsc.subcore_barrier()

    def main_body(t, carry):
        for b in range(GRP):
            i = t * GRP + b

            @pl.when(i + GDEPTH + 2 < CPT)
            def _():
                idx_load(i + GDEPTH + 2, lax.rem(i + GDEPTH + 2, NIDX))

            gather_wait()
            scatter(lax.rem(i, NIDX), lax.rem(i, RING))

            @pl.when(i >= 3)
            def _():
                scatter_wait()

            @pl.when(i + GDEPTH < CPT)
            def _():
                idx_wait()
                gather(i + GDEPTH, lax.rem(i + GDEPTH, NIDX),
                       lax.rem(i + GDEPTH, RING))
        return carry
    lax.fori_loop(0, N_GRP, main_body, 0)

    for _ in range(3):
        scatter_wait()

    plsc.subcore_barrier()

    def wb_in(i):
        k = s + i * NS
        pltpu.async_copy(agg_sh.at[pl.ds(k * CHUNK, CHUNK)],
                         rows_v.at[i % RING], sg)
        pltpu.async_copy(deg_sh.at[pl.ds(k * CHUNK, CHUNK)],
                         degb_v.at[i, pl.ds(0, CHUNK)], si)

    def wb_in_wait(i):
        k = s + i * NS
        pltpu.make_async_copy(agg_sh.at[pl.ds(k * CHUNK, CHUNK)],
                              rows_v.at[i % RING], sg).wait()
        pltpu.make_async_copy(deg_sh.at[pl.ds(k * CHUNK, CHUNK)],
                              degb_v.at[i, pl.ds(0, CHUNK)], si).wait()

    def wb_out(i):
        k = s + i * NS
        pltpu.async_copy(rows_v.at[i % RING],
                         agg_out.at[c, pl.ds(k * CHUNK, CHUNK)], ss)
        pltpu.async_copy(degb_v.at[i, pl.ds(0, CHUNK)],
                         deg_out.at[pl.ds(c * N_NODES + k * CHUNK, CHUNK)],
                         sd)

    def wb_out_wait(i):
        k = s + i * NS
        pltpu.make_async_copy(rows_v.at[i % RING],
                              agg_out.at[c, pl.ds(k * CHUNK, CHUNK)],
                              ss).wait()
        pltpu.make_async_copy(
            degb_v.at[i, pl.ds(0, CHUNK)],
            deg_out.at[pl.ds(c * N_NODES + k * CHUNK, CHUNK)], sd).wait()

    last_ok = s + (WB_STEPS - 1) * NS < ROW_CHUNKS
    for i in range(WB_STEPS):
        if i >= RING:
            wb_out_wait(i - RING)
        if i < WB_STEPS - 1:
            wb_in(i)
        else:
            @pl.when(last_ok)
            def _():
                wb_in(i)
        if i >= 1:
            wb_in_wait(i - 1)
            wb_out(i - 1)

    @pl.when(last_ok)
    def _():
        wb_in_wait(WB_STEPS - 1)
        wb_out(WB_STEPS - 1)
    for i in range(WB_STEPS - RING, WB_STEPS - 1):
        wb_out_wait(i)

    @pl.when(last_ok)
    def _():
        wb_out_wait(WB_STEPS - 1)


_sc_aggregate = functools.partial(
    pl.kernel,
    out_type=(jax.ShapeDtypeStruct((NC, N_NODES, IN_CH), jnp.float32),
              jax.ShapeDtypeStruct((NC * N_NODES,), jnp.float32)),
    mesh=plsc.VectorSubcoreMesh(core_axis_name="c", subcore_axis_name="s"),
    scratch_types=[
        pltpu.VMEM((NIDX, CHUNK), jnp.int32),
        pltpu.VMEM((NIDX, CHUNK), jnp.int32),
        pltpu.VMEM((RING, CHUNK, IN_CH), jnp.float32),
        pltpu.VMEM((48,), jnp.float32),
        pltpu.VMEM((48,), jnp.float32),
        pltpu.VMEM((WB_STEPS, 48), jnp.float32),
        pltpu.VMEM_SHARED((N_NODES, IN_CH), jnp.float32),
        pltpu.VMEM_SHARED((N_NODES,), jnp.float32),
        pltpu.SemaphoreType.DMA,
        pltpu.SemaphoreType.DMA,
        pltpu.SemaphoreType.DMA,
        pltpu.SemaphoreType.DMA,
    ],
)(_sc_body)


ROW_BLK = 1000


_DN_T = (((1,), (1,)), ((), ()))


def _tc_body(agg_ref, deg_ref, wmsg_ref, bmsg_ref, wupd_ref, bupd_ref,
             out_ref):
    a = agg_ref[0] + agg_ref[1]
    d = deg_ref[0] + deg_ref[1]
    m = lax.dot_general(a, wmsg_ref[...], _DN_T,
                        preferred_element_type=jnp.float32)
    m = m + d * bmsg_ref[...]
    u = lax.dot_general(m, wupd_ref[...], _DN_T,
                        preferred_element_type=jnp.float32)
    out_ref[...] = u + bupd_ref[...]


_tc_update = pl.pallas_call(
    _tc_body,
    grid=(N_NODES // ROW_BLK,),
    in_specs=[
        pl.BlockSpec((NC, ROW_BLK, IN_CH), lambda i: (0, i, 0)),
        pl.BlockSpec((NC, ROW_BLK, 1), lambda i: (0, i, 0)),
        pl.BlockSpec((IN_CH, OUT_CH), lambda i: (0, 0)),
        pl.BlockSpec((1, OUT_CH), lambda i: (0, 0)),
        pl.BlockSpec((OUT_CH, OUT_CH), lambda i: (0, 0)),
        pl.BlockSpec((1, OUT_CH), lambda i: (0, 0)),
    ],
    out_specs=pl.BlockSpec((ROW_BLK, OUT_CH), lambda i: (i, 0)),
    out_shape=jax.ShapeDtypeStruct((N_NODES, OUT_CH), jnp.float32),
)


@jax.jit
def kernel(x, edge_index, W_msg, b_msg, W_upd, b_upd):
    ei = edge_index.astype(jnp.int32).reshape(2 * N_EDGES)
    agg, deg = _sc_aggregate(x, ei)
    return _tc_update(agg, deg.reshape(NC, N_NODES, 1),
                      W_msg, b_msg.reshape(1, OUT_CH),
                      W_upd, b_upd.reshape(1, OUT_CH))
